# Initial kernel scaffold; baseline (speedup 1.0000x reference)
import jax
import jax.numpy as jnp
from jax.experimental import pallas as pl


def kernel(x, y, patch_w, patch_b, cls_token, pos_embed, layers, dec_w, dec_b):
    raise NotImplementedError("write your pallas kernel here")


# speedup vs baseline: 44.2649x; 44.2649x over previous
import functools
import math

import jax
import jax.numpy as jnp
from jax.experimental import pallas as pl

EMSIZE = 768
NHEADS = 12
NHID = 3072
N_EXPERT = 8
IMG = 32
PATCH = 4
SEQLEN = (IMG // PATCH) * (IMG // PATCH)
HEAD_DIM = EMSIZE // NHEADS


def _mm_body(x_ref, w_ref, b_ref, o_ref, *, relu):
    acc = jnp.dot(x_ref[...], w_ref[...], preferred_element_type=jnp.float32)
    acc = acc + b_ref[...]
    if relu:
        acc = jnp.maximum(acc, 0.0)
    o_ref[...] = acc


def _mm(x, w, b, *, bm, relu=False):
    m, k = x.shape
    n = w.shape[1]
    grid = (m // bm,)
    return pl.pallas_call(
        functools.partial(_mm_body, relu=relu),
        grid=grid,
        in_specs=[
            pl.BlockSpec((bm, k), lambda i: (i, 0)),
            pl.BlockSpec((k, n), lambda i: (0, 0)),
            pl.BlockSpec((1, n), lambda i: (0, 0)),
        ],
        out_specs=pl.BlockSpec((bm, n), lambda i: (i, 0)),
        out_shape=jax.ShapeDtypeStruct((m, n), jnp.float32),
    )(x, w, b.reshape(1, n))


def _ln(v, g, b, eps=1e-5):
    mu = jnp.mean(v, axis=-1, keepdims=True)
    var = jnp.mean((v - mu) ** 2, axis=-1, keepdims=True)
    return (v - mu) * jax.lax.rsqrt(var + eps) * g + b


def _mm_res_ln_body(x_ref, w_ref, b_ref, r_ref, g_ref, bb_ref, o_ref):
    acc = jnp.dot(x_ref[...], w_ref[...], preferred_element_type=jnp.float32)
    v = acc + b_ref[...] + r_ref[...]
    o_ref[...] = _ln(v, g_ref[...], bb_ref[...])


def _mm_res_ln(x, w, b, res, g, beta, *, bm):
    m, k = x.shape
    n = w.shape[1]
    grid = (m // bm,)
    return pl.pallas_call(
        _mm_res_ln_body,
        grid=grid,
        in_specs=[
            pl.BlockSpec((bm, k), lambda i: (i, 0)),
            pl.BlockSpec((k, n), lambda i: (0, 0)),
            pl.BlockSpec((1, n), lambda i: (0, 0)),
            pl.BlockSpec((bm, n), lambda i: (i, 0)),
            pl.BlockSpec((1, n), lambda i: (0, 0)),
            pl.BlockSpec((1, n), lambda i: (0, 0)),
        ],
        out_specs=pl.BlockSpec((bm, n), lambda i: (i, 0)),
        out_shape=jax.ShapeDtypeStruct((m, n), jnp.float32),
    )(x, w, b.reshape(1, n), res, g.reshape(1, n), beta.reshape(1, n))


def _attn_body(q_ref, k_ref, v_ref, o_ref):
    q = q_ref[0]
    k = k_ref[0]
    v = v_ref[0]
    s = jax.lax.dot_general(
        q, k, (((2,), (2,)), ((0,), (0,))),
        preferred_element_type=jnp.float32)
    s = s * (1.0 / math.sqrt(HEAD_DIM))
    m = jnp.max(s, axis=-1, keepdims=True)
    e = jnp.exp(s - m)
    p = e / jnp.sum(e, axis=-1, keepdims=True)
    o = jax.lax.dot_general(
        p, v, (((2,), (1,)), ((0,), (0,))),
        preferred_element_type=jnp.float32)
    o_ref[0] = o


def _attention(q, k, v, *, bb):
    B, H, S, Dh = q.shape
    grid = (B // bb,)
    spec = pl.BlockSpec((bb, H, S, Dh), lambda i: (i, 0, 0, 0))
    return pl.pallas_call(
        _attn_body,
        grid=grid,
        in_specs=[spec, spec, spec],
        out_specs=spec,
        out_shape=jax.ShapeDtypeStruct((B, H, S, Dh), jnp.float32),
    )(q, k, v)


def _mha(x_flat, B, S, layer, *, bm):
    wqkv = jnp.concatenate([layer['wq'], layer['wk'], layer['wv']], axis=1)
    bqkv = jnp.concatenate([layer['bq'], layer['bk'], layer['bv']], axis=0)
    qkv = _mm(x_flat, wqkv, bqkv, bm=bm)
    qkv = qkv.reshape(B, S, 3, NHEADS, HEAD_DIM).transpose(2, 0, 3, 1, 4)
    o = _attention(qkv[0], qkv[1], qkv[2], bb=1)
    o = o.transpose(0, 2, 1, 3).reshape(B * S, EMSIZE)
    return _mm_res_ln(o, layer['wo'], layer['bo'], x_flat,
                      layer['ln1_g'], layer['ln1_b'], bm=bm)


def _gate_body(x_ref, w_ref, o_ref):
    logits = jnp.dot(x_ref[...], w_ref[...], preferred_element_type=jnp.float32)
    m = jnp.max(logits, axis=-1, keepdims=True)
    e = jnp.exp(logits - m)
    p = e / jnp.sum(e, axis=-1, keepdims=True)
    iota = jax.lax.broadcasted_iota(jnp.int32, p.shape, 1)
    m1 = jnp.max(p, axis=-1, keepdims=True)
    idx1 = jnp.min(jnp.where(p == m1, iota, N_EXPERT), axis=-1, keepdims=True)
    mask1 = iota == idx1
    p2 = jnp.where(mask1, -jnp.inf, p)
    m2 = jnp.max(p2, axis=-1, keepdims=True)
    idx2 = jnp.min(jnp.where(p2 == m2, iota, N_EXPERT), axis=-1, keepdims=True)
    mask2 = iota == idx2
    cw = jnp.where(mask1 | mask2, p, 0.0) / (m1 + m2)
    o_ref[...] = cw


def _gate(x, gate_w, *, bm):
    m, k = x.shape
    n = gate_w.shape[1]
    return pl.pallas_call(
        _gate_body,
        grid=(m // bm,),
        in_specs=[
            pl.BlockSpec((bm, k), lambda i: (i, 0)),
            pl.BlockSpec((k, n), lambda i: (0, 0)),
        ],
        out_specs=pl.BlockSpec((bm, n), lambda i: (i, 0)),
        out_shape=jax.ShapeDtypeStruct((m, n), jnp.float32),
    )(x, gate_w)


def _moe_body(x_ref, w1_ref, b1_ref, w2_ref, b2_ref, cw_ref, g_ref, bb_ref,
              o_ref):
    e = pl.program_id(1)
    h = jnp.dot(x_ref[...], w1_ref[0], preferred_element_type=jnp.float32)
    h = jnp.maximum(h + b1_ref[0], 0.0)
    y = jnp.dot(h, w2_ref[0], preferred_element_type=jnp.float32) + b2_ref[0]
    cw = cw_ref[...]
    iota = jax.lax.broadcasted_iota(jnp.int32, cw.shape, 1)
    w = jnp.sum(jnp.where(iota == e, cw, 0.0), axis=1, keepdims=True)
    contrib = w * y

    @pl.when(e == 0)
    def _init():
        o_ref[...] = contrib

    @pl.when(e > 0)
    def _acc():
        o_ref[...] += contrib

    @pl.when(e == N_EXPERT - 1)
    def _fin():
        v = o_ref[...] + x_ref[...]
        o_ref[...] = _ln(v, g_ref[...], bb_ref[...])


def _moe(x, layer, *, bm):
    m, d = x.shape
    cw = _gate(x, layer['gate_w'], bm=bm)
    grid = (m // bm, N_EXPERT)
    return pl.pallas_call(
        _moe_body,
        grid=grid,
        in_specs=[
            pl.BlockSpec((bm, d), lambda i, e: (i, 0)),
            pl.BlockSpec((1, d, NHID), lambda i, e: (e, 0, 0)),
            pl.BlockSpec((1, 1, NHID), lambda i, e: (e, 0, 0)),
            pl.BlockSpec((1, NHID, d), lambda i, e: (e, 0, 0)),
            pl.BlockSpec((1, 1, d), lambda i, e: (e, 0, 0)),
            pl.BlockSpec((bm, N_EXPERT), lambda i, e: (i, 0)),
            pl.BlockSpec((1, d), lambda i, e: (0, 0)),
            pl.BlockSpec((1, d), lambda i, e: (0, 0)),
        ],
        out_specs=pl.BlockSpec((bm, d), lambda i, e: (i, 0)),
        out_shape=jax.ShapeDtypeStruct((m, d), jnp.float32),
    )(x, layer['exp_w1'], layer['exp_b1'].reshape(N_EXPERT, 1, NHID),
      layer['exp_w2'], layer['exp_b2'].reshape(N_EXPERT, 1, d), cw,
      layer['ln2_g'].reshape(1, d), layer['ln2_b'].reshape(1, d))


def _head_body(x_ref, w_ref, b_ref, y_ref, o_ref):
    logits = jnp.dot(x_ref[...], w_ref[...], preferred_element_type=jnp.float32)
    logits = logits + b_ref[...]
    m = jnp.max(logits, axis=-1, keepdims=True)
    lse = m + jnp.log(jnp.sum(jnp.exp(logits - m), axis=-1, keepdims=True))
    iota = jax.lax.broadcasted_iota(jnp.int32, logits.shape, 1)
    onehot = iota == y_ref[...]
    picked = jnp.sum(jnp.where(onehot, logits, 0.0), axis=-1, keepdims=True)
    loss = -jnp.sum(picked - lse, axis=0, keepdims=True)
    o_ref[...] = loss


def _head(cls_out, dec_w, dec_b, y):
    B, d = cls_out.shape
    C = dec_w.shape[1]
    out = pl.pallas_call(
        _head_body,
        in_specs=[
            pl.BlockSpec((B, d), lambda: (0, 0)),
            pl.BlockSpec((d, C), lambda: (0, 0)),
            pl.BlockSpec((1, C), lambda: (0, 0)),
            pl.BlockSpec((B, 1), lambda: (0, 0)),
        ],
        out_specs=pl.BlockSpec((1, 1), lambda: (0, 0)),
        out_shape=jax.ShapeDtypeStruct((1, 1), jnp.float32),
    )(cls_out, dec_w, dec_b.reshape(1, C), y.astype(jnp.int32).reshape(B, 1))
    return out.reshape(())


def kernel(x, y, patch_w, patch_b, cls_token, pos_embed, layers, dec_w, dec_b):
    B = x.shape[0]
    p = IMG // PATCH
    S = SEQLEN + 1
    patches = x.reshape(B, 3, p, PATCH, p, PATCH).transpose(
        0, 2, 4, 1, 3, 5).reshape(B * p * p, 3 * PATCH * PATCH)
    hp = _mm(patches, patch_w, patch_b, bm=512)
    hp = hp.reshape(B, p * p, EMSIZE)
    cls = jnp.broadcast_to(cls_token, (B, 1, EMSIZE))
    h = jnp.concatenate([cls, hp], axis=1) + pos_embed

    bm = (B * S) // 8
    h = h.reshape(B * S, EMSIZE)
    for i, layer in enumerate(layers):
        h = _mha(h, B, S, layer, bm=bm)
        if i % 2 == 0:
            ff = _mm(h, layer['ff_w1'], layer['ff_b1'], bm=bm, relu=True)
            h = _mm_res_ln(ff, layer['ff_w2'], layer['ff_b2'], h,
                           layer['ln2_g'], layer['ln2_b'], bm=bm)
        else:
            h = _moe(h, layer, bm=bm)

    cls_out = h.reshape(B, S, EMSIZE)[:, 0, :]
    return _head(cls_out, dec_w, dec_b, y)
